# Initial kernel scaffold; baseline (speedup 1.0000x reference)
import jax
import jax.numpy as jnp
from jax.experimental import pallas as pl


def kernel(pred_porb, target_porb, GD_stat_ema):
    raise NotImplementedError("write your pallas kernel here")


# speedup vs baseline: 17.5995x; 17.5995x over previous
import functools

import jax
import jax.numpy as jnp
from jax import lax
from jax.experimental import pallas as pl
from jax.experimental.pallas import tpu as pltpu
from jax.experimental.pallas import tpu_sc as plsc

_NUM_BINS = 10
_ALPHA = 1.0 - 1e-06
_N_ROWS = 16384
_N_COLS = 1024
_N = _N_ROWS * _N_COLS
_NC = 2
_NS = 16
_NW = _NC * _NS
_PER_W = _N // _NW
_CHUNK = 16384
_NCHUNK = _PER_W // _CHUNK
_VECS = _CHUNK // 16
_LN2 = 0.6931471805599453


def _ln(x):
    ix = plsc.bitcast(x, jnp.int32)
    e = lax.shift_right_arithmetic(ix - 0x3F3504F3, 23)
    m = plsc.bitcast(ix - lax.shift_left(e, 23), jnp.float32)
    ef = lax.convert_element_type(e, jnp.float32)
    s = (m - 1.0) / (m + 1.0)
    z = s * s
    poly = 1.0 + z * (1.0 / 3.0 + z * (0.2 + z * (1.0 / 7.0)))
    return ef * _LN2 + (s + s) * poly


def _sc_body(pred_hbm, targ_hbm, ema_hbm, loss_out, cnt_out,
             pbuf, tbuf, lossb, cntb, wbuf,
             semp0, semp1, semt0, semt1):
    wid = lax.axis_index("s") * _NC + lax.axis_index("c")
    base = wid * _PER_W

    zf = jnp.zeros((16,), jnp.float32)
    onef = jnp.ones((16,), jnp.float32)
    lane = lax.iota(jnp.int32, 16)
    for b in range(_NUM_BINS):
        lossb[pl.ds(b * 16, 16)] = zf
        cntb[pl.ds(b * 16, 16)] = zf

    pltpu.sync_copy(ema_hbm, wbuf)
    wbuf[...] = 1.0 / wbuf[...] + 0.001

    pb = [pbuf.at[pl.ds(0, _CHUNK)], pbuf.at[pl.ds(_CHUNK, _CHUNK)]]
    tb = [tbuf.at[pl.ds(0, _CHUNK)], tbuf.at[pl.ds(_CHUNK, _CHUNK)]]
    semp = [semp0, semp1]
    semt = [semt0, semt1]

    def compute_chunk(pref, tref):
        def body(i, carry):
            off = lax.shift_left(i, 4)
            p = pref[pl.ds(off, 16)]
            t = tref[pl.ds(off, 16)]
            q = 1.0 - p
            lp = jnp.where(p > 0.0, _ln(p), -100.0)
            lq = jnp.where(q > 0.0, _ln(q), -100.0)
            raw = t * (lq - lp) - lq
            gm = jnp.abs(p - t)
            idx = lax.convert_element_type(gm * 10.0, jnp.int32)
            idx = jnp.minimum(idx, _NUM_BINS - 1)
            sidx = lax.shift_left(idx, 4) + lane
            plsc.addupdate_scatter(lossb, [sidx], raw)
            plsc.addupdate_scatter(cntb, [sidx], onef)
            return carry
        lax.fori_loop(0, _VECS, body, 0, unroll=4)

    npairs = _NCHUNK // 2
    pltpu.async_copy(pred_hbm.at[pl.ds(base, _CHUNK)], pb[0], semp[0])
    pltpu.async_copy(targ_hbm.at[pl.ds(base, _CHUNK)], tb[0], semt[0])

    def pair_body(j, carry):
        off1 = base + (2 * j + 1) * _CHUNK
        h1p = pltpu.async_copy(pred_hbm.at[pl.ds(off1, _CHUNK)], pb[1], semp[1])
        h1t = pltpu.async_copy(targ_hbm.at[pl.ds(off1, _CHUNK)], tb[1], semt[1])
        pltpu.make_async_copy(
            pred_hbm.at[pl.ds(base, _CHUNK)], pb[0], semp[0]).wait()
        pltpu.make_async_copy(
            targ_hbm.at[pl.ds(base, _CHUNK)], tb[0], semt[0]).wait()
        compute_chunk(pb[0], tb[0])

        @pl.when(j < npairs - 1)
        def _():
            off0 = base + (2 * j + 2) * _CHUNK
            pltpu.async_copy(pred_hbm.at[pl.ds(off0, _CHUNK)], pb[0], semp[0])
            pltpu.async_copy(targ_hbm.at[pl.ds(off0, _CHUNK)], tb[0], semt[0])

        h1p.wait()
        h1t.wait()
        compute_chunk(pb[1], tb[1])
        return carry

    lax.fori_loop(0, npairs, pair_body, 0)

    acc = zf
    for b in range(_NUM_BINS):
        wb = plsc.load_gather(wbuf, [jnp.full((16,), b, jnp.int32)])
        acc = acc + wb * lossb[pl.ds(b * 16, 16)]
    wbuf[...] = acc
    pltpu.sync_copy(wbuf, loss_out.at[wid])
    pltpu.sync_copy(cntb, cnt_out.at[wid])


@jax.jit
def _sc_launch(pred_flat, targ_flat, ema16):
    kfn = pl.kernel(
        _sc_body,
        out_type=(
            jax.ShapeDtypeStruct((_NW, 16), jnp.float32),
            jax.ShapeDtypeStruct((_NW, _NUM_BINS * 16), jnp.float32),
        ),
        mesh=plsc.VectorSubcoreMesh(core_axis_name="c", subcore_axis_name="s"),
        compiler_params=pltpu.CompilerParams(needs_layout_passes=False),
        scratch_types=[
            pltpu.VMEM((2 * _CHUNK,), jnp.float32),
            pltpu.VMEM((2 * _CHUNK,), jnp.float32),
            pltpu.VMEM((_NUM_BINS * 16,), jnp.float32),
            pltpu.VMEM((_NUM_BINS * 16,), jnp.float32),
            pltpu.VMEM((16,), jnp.float32),
            pltpu.SemaphoreType.DMA,
            pltpu.SemaphoreType.DMA,
            pltpu.SemaphoreType.DMA,
            pltpu.SemaphoreType.DMA,
        ],
    )
    return kfn(pred_flat, targ_flat, ema16)


def kernel(pred_porb, target_porb, GD_stat_ema):
    pred_flat = pred_porb.reshape(-1)
    targ_flat = target_porb.reshape(-1)
    ema16 = jnp.concatenate(
        [GD_stat_ema, jnp.ones((16 - _NUM_BINS,), jnp.float32)])
    loss_part, cnt_part = _sc_launch(pred_flat, targ_flat, ema16)
    loss_final = jnp.sum(loss_part) / jnp.float32(_N)
    counts = cnt_part.reshape(_NW, _NUM_BINS, 16).sum(axis=(0, 2))
    hist = counts / (jnp.sum(counts) + 1e-10) * _NUM_BINS
    ema = GD_stat_ema * _ALPHA + (1.0 - _ALPHA) * hist
    ema = ema / (jnp.sum(ema) + 1e-10) * _NUM_BINS
    return loss_final, ema
